# baseline (speedup 1.0000x reference)
import functools

import jax
import jax.numpy as jnp
from jax import lax
from jax.experimental import pallas as pl
from jax.experimental.pallas import tpu as pltpu
from jax.experimental.pallas import tpu_sc as plsc

VOCAB = 100000
EMBED = 32
BLOCK = 8
BATCH = 128
ROWS = BATCH * BLOCK

NC = 2
NS = 16
NW = NC * NS
ROWS_PER_W = ROWS // NW

VT = 2048
NVT = (VOCAB + VT - 1) // VT


def _emb_body(tok_hbm, idx_hbm, pos_hbm, out_hbm, idx_v, rows_v, pos_v, sem):
    wid = lax.axis_index("s") * NC + lax.axis_index("c")
    base = wid * ROWS_PER_W
    pltpu.sync_copy(idx_hbm.at[pl.ds(base, ROWS_PER_W)], idx_v)
    pltpu.sync_copy(pos_hbm, pos_v)
    pltpu.async_copy(tok_hbm.at[idx_v], rows_v, sem).wait()
    for r in range(ROWS_PER_W):
        for c in range(EMBED // 16):
            sl = pl.ds(c * 16, 16)
            rows_v[r, sl] = rows_v[r, sl] + pos_v[r % BLOCK, sl]
    pltpu.sync_copy(rows_v, out_hbm.at[pl.ds(base, ROWS_PER_W)])


@functools.cache
def _emb_kernel():
    return pl.kernel(
        _emb_body,
        out_type=jax.ShapeDtypeStruct((ROWS, EMBED), jnp.float32),
        mesh=plsc.VectorSubcoreMesh(
            core_axis_name="c", subcore_axis_name="s", num_cores=NC, num_subcores=NS
        ),
        scratch_types=[
            pltpu.VMEM((ROWS_PER_W,), jnp.int32),
            pltpu.VMEM((ROWS_PER_W, EMBED), jnp.float32),
            pltpu.VMEM((BLOCK, EMBED), jnp.float32),
            pltpu.SemaphoreType.DMA,
        ],
        compiler_params=pltpu.CompilerParams(use_tc_tiling_on_sc=False),
    )


def _mm_body(x_ref, w_ref, b_ref, out_ref):
    out_ref[...] = (
        jnp.dot(x_ref[...], w_ref[...], preferred_element_type=jnp.float32)
        + b_ref[...]
    )


def kernel(idx, tok_table, pos_table, W, b):
    idx_flat = idx.reshape(ROWS).astype(jnp.int32)
    x = _emb_kernel()(tok_table, idx_flat, pos_table)
    logits = pl.pallas_call(
        _mm_body,
        grid=(NVT,),
        in_specs=[
            pl.BlockSpec((ROWS, EMBED), lambda j: (0, 0)),
            pl.BlockSpec((EMBED, VT), lambda j: (0, j)),
            pl.BlockSpec((1, VT), lambda j: (0, j)),
        ],
        out_specs=pl.BlockSpec((ROWS, VT), lambda j: (0, j)),
        out_shape=jax.ShapeDtypeStruct((ROWS, VOCAB), jnp.float32),
    )(x, W, b.reshape(1, VOCAB))
    return logits.reshape(BATCH, BLOCK, VOCAB)
